# baseline (speedup 1.0000x reference)
import functools
import itertools

import jax
import jax.numpy as jnp
import numpy as np
from jax import lax
from jax.experimental import pallas as pl
from jax.experimental.pallas import tpu as pltpu
from jax.experimental.pallas import tpu_sc as plsc


def _combo_pairs():
    cola = [set(c) for c in itertools.combinations(range(8), 2)]
    pairs = []
    for i, si in enumerate(cola):
        for j, sj in enumerate(cola):
            if not si.intersection(sj):
                if [i, j] not in pairs and [j, i] not in pairs:
                    pairs.append([i, j])
    return np.array(sorted(pairs), dtype=np.int32)


_PAIRS = _combo_pairs()

_NROWS = 131072
_NCOL = 28
_NCOMB = 210
_NC = 2
_NS = 16
_NW = _NC * _NS
_ROWS_PER_W = _NROWS // _NW
_CH = 128
_NCHUNK = _ROWS_PER_W // _CH
_L = 16
_NK = 14

_PADDED = list(range(_NCOMB)) + list(range(_NK * _L - _NCOMB))
_ACOL = np.array(
    [[int(_PAIRS[_PADDED[k * _L + l], 0]) for l in range(_L)] for k in range(_NK)],
    dtype=np.int32,
)
_BCOL = np.array(
    [[int(_PAIRS[_PADDED[k * _L + l], 1]) for l in range(_L)] for k in range(_NK)],
    dtype=np.int32,
)
_TAILCOL = np.array(
    [(_NK - 1) * _L + l if l < 2 else 0 for l in range(_L)], dtype=np.int32
)


def _compute_chunk(xbuf, outbuf, ridxbuf, rbase, tabbuf):
    acols = tuple(tabbuf[k, :] for k in range(_NK))
    bcols = tuple(tabbuf[_NK + k, :] for k in range(_NK))
    tailcol = tabbuf[2 * _NK, :]
    tailmask = lax.iota(jnp.int32, _L) < 2
    zero = jnp.float32(0.0)
    inf = jnp.float32(np.inf)

    def row_body(r, carry):
        acols, bcols, tailcol = carry
        rvec = jnp.broadcast_to(r, (_L,)).astype(jnp.int32)
        allnan = None
        for k in range(_NK):
            va = plsc.load_gather(xbuf, [rvec, acols[k]])
            vb = plsc.load_gather(xbuf, [rvec, bcols[k]])
            s = va + vb
            z = s == zero
            v = jnp.where(z, inf, jnp.abs(va - vb) / s)
            allnan = z if allnan is None else (allnan & z)
            if k < _NK - 1:
                outbuf[r, pl.ds(k * _L, _L)] = v
            else:
                plsc.store_scatter(outbuf, [rvec, tailcol], v, mask=tailmask)

        @pl.when(jnp.all(allnan))
        def _fix():
            colvec = plsc.load_gather(ridxbuf, [rvec + rbase])
            plsc.store_scatter(
                outbuf,
                [rvec, colvec],
                jnp.full((_L,), 1.0, jnp.float32),
                mask=lax.iota(jnp.int32, _L) < 1,
            )

        return carry

    lax.fori_loop(0, _CH, row_body, (acols, bcols, tailcol), unroll=2)


def _body(
    x_hbm, ridx_hbm, tab_hbm, out_hbm,
    xb0, xb1, ob0, ob1, ridxbuf, tabbuf,
    isem0, isem1, osem0, osem1,
):
    wid = lax.axis_index("s") * _NC + lax.axis_index("c")
    w_base = wid * _ROWS_PER_W
    pltpu.sync_copy(tab_hbm, tabbuf)
    pltpu.sync_copy(ridx_hbm.at[pl.ds(w_base, _ROWS_PER_W)], ridxbuf)

    xbufs = (xb0, xb1)
    obufs = (ob0, ob1)
    isems = (isem0, isem1)
    osems = (osem0, osem1)

    def in_copy(ci, p):
        return pltpu.make_async_copy(
            x_hbm.at[pl.ds(w_base + ci * _CH, _CH)], xbufs[p], isems[p]
        )

    def out_copy(ci, p):
        return pltpu.make_async_copy(
            obufs[p],
            out_hbm.at[pl.ds(w_base + ci * _CH, _CH)],
            osems[p],
        )

    in_copy(0, 0).start()
    in_copy(1, 1).start()

    def loop_body(k, _):
        for p in range(2):
            ci = 2 * k + p
            in_copy(ci, p).wait()

            @pl.when(ci >= 2)
            def _wait_out():
                out_copy(ci - 2, p).wait()

            _compute_chunk(xbufs[p], obufs[p], ridxbuf, ci * _CH, tabbuf)
            out_copy(ci, p).start()

            @pl.when(ci + 2 < _NCHUNK)
            def _prefetch():
                in_copy(ci + 2, p).start()
        return ()

    lax.fori_loop(0, _NCHUNK // 2, loop_body, (), unroll=False)

    out_copy(_NCHUNK - 2, 0).wait()
    out_copy(_NCHUNK - 1, 1).wait()


@jax.jit
def kernel(x):
    ridx = jax.random.randint(
        jax.random.key(1), (x.shape[0],), 0, _NCOMB
    ).astype(jnp.int32)
    tab = jnp.asarray(
        np.concatenate([_ACOL, _BCOL, _TAILCOL[None, :]], axis=0)
    )
    run = pl.kernel(
        _body,
        out_type=jax.ShapeDtypeStruct((_NROWS, _NCOMB), jnp.float32),
        mesh=plsc.VectorSubcoreMesh(core_axis_name="c", subcore_axis_name="s"),
        scratch_types=[
            pltpu.VMEM((_CH, _NCOL), jnp.float32),
            pltpu.VMEM((_CH, _NCOL), jnp.float32),
            pltpu.VMEM((_CH, _NCOMB), jnp.float32),
            pltpu.VMEM((_CH, _NCOMB), jnp.float32),
            pltpu.VMEM((_ROWS_PER_W,), jnp.int32),
            pltpu.VMEM((2 * _NK + 1, _L), jnp.int32),
            pltpu.SemaphoreType.DMA,
            pltpu.SemaphoreType.DMA,
            pltpu.SemaphoreType.DMA,
            pltpu.SemaphoreType.DMA,
        ],
        compiler_params=pltpu.CompilerParams(needs_layout_passes=False),
    )
    return run(x, ridx, tab)


# speedup vs baseline: 2.4903x; 2.4903x over previous
import functools
import itertools

import jax
import jax.numpy as jnp
import numpy as np
from jax import lax
from jax.experimental import pallas as pl
from jax.experimental.pallas import tpu as pltpu
from jax.experimental.pallas import tpu_sc as plsc


def _combo_pairs():
    cola = [set(c) for c in itertools.combinations(range(8), 2)]
    pairs = []
    for i, si in enumerate(cola):
        for j, sj in enumerate(cola):
            if not si.intersection(sj):
                if [i, j] not in pairs and [j, i] not in pairs:
                    pairs.append([i, j])
    return np.array(sorted(pairs), dtype=np.int32)


_PAIRS = _combo_pairs()

_NROWS = 131072
_NCOL = 28
_NCOMB = 210
_NC = 2
_NS = 16
_NW = _NC * _NS
_ROWS_PER_W = _NROWS // _NW
_CH = 128
_NCHUNK = _ROWS_PER_W // _CH
_L = 16
_NK = 14

_PADDED = list(range(_NCOMB)) + list(range(_NK * _L - _NCOMB))
_ACOL = np.array(
    [[int(_PAIRS[_PADDED[k * _L + l], 0]) for l in range(_L)] for k in range(_NK)],
    dtype=np.int32,
)
_BCOL = np.array(
    [[int(_PAIRS[_PADDED[k * _L + l], 1]) for l in range(_L)] for k in range(_NK)],
    dtype=np.int32,
)
_TAILCOL = np.array(
    [(_NK - 1) * _L + l if l < 2 else 0 for l in range(_L)], dtype=np.int32
)


def _compute_chunk(xbuf, outbuf, ridxbuf, rbase, tabbuf):
    acols = tuple(tabbuf[k, :] for k in range(_NK))
    bcols = tuple(tabbuf[_NK + k, :] for k in range(_NK))
    tailcol = tabbuf[2 * _NK, :]
    tailmask = lax.iota(jnp.int32, _L) < 2
    zero = jnp.float32(0.0)
    inf = jnp.float32(np.inf)

    def row_body(r, carry):
        acols, bcols, tailcol = carry
        rvec = jnp.broadcast_to(r, (_L,)).astype(jnp.int32)
        vs = []
        allnan = None
        for k in range(_NK):
            va = plsc.load_gather(xbuf, [rvec, acols[k]])
            vb = plsc.load_gather(xbuf, [rvec, bcols[k]])
            s = va + vb
            z = s == zero
            v = jnp.where(z, inf, jnp.abs(va - vb) / s)
            allnan = z if allnan is None else (allnan & z)
            vs.append(v)
        for k in range(_NK - 1):
            outbuf[r, pl.ds(k * _L, _L)] = vs[k]
        plsc.store_scatter(outbuf, [rvec, tailcol], vs[_NK - 1], mask=tailmask)

        @pl.when(jnp.all(allnan))
        def _fix():
            colvec = plsc.load_gather(ridxbuf, [rvec + rbase])
            plsc.store_scatter(
                outbuf,
                [rvec, colvec],
                jnp.full((_L,), 1.0, jnp.float32),
                mask=lax.iota(jnp.int32, _L) < 1,
            )

        return carry

    lax.fori_loop(0, _CH, row_body, (acols, bcols, tailcol), unroll=4)


def _body(
    x_hbm, ridx_hbm, tab_hbm, out_hbm,
    xb0, xb1, ob0, ob1, ridxbuf, tabbuf,
    isem0, isem1, osem0, osem1,
):
    wid = lax.axis_index("s") * _NC + lax.axis_index("c")
    w_base = wid * _ROWS_PER_W
    pltpu.sync_copy(tab_hbm, tabbuf)
    pltpu.sync_copy(ridx_hbm.at[pl.ds(w_base, _ROWS_PER_W)], ridxbuf)

    xbufs = (xb0, xb1)
    obufs = (ob0, ob1)
    isems = (isem0, isem1)
    osems = (osem0, osem1)

    def in_copy(ci, p):
        return pltpu.make_async_copy(
            x_hbm.at[pl.ds(w_base + ci * _CH, _CH)], xbufs[p], isems[p]
        )

    def out_copy(ci, p):
        return pltpu.make_async_copy(
            obufs[p],
            out_hbm.at[pl.ds(w_base + ci * _CH, _CH)],
            osems[p],
        )

    in_copy(0, 0).start()
    in_copy(1, 1).start()

    def loop_body(k, _):
        for p in range(2):
            ci = 2 * k + p
            in_copy(ci, p).wait()

            @pl.when(ci >= 2)
            def _wait_out():
                out_copy(ci - 2, p).wait()

            _compute_chunk(xbufs[p], obufs[p], ridxbuf, ci * _CH, tabbuf)
            out_copy(ci, p).start()

            @pl.when(ci + 2 < _NCHUNK)
            def _prefetch():
                in_copy(ci + 2, p).start()
        return ()

    lax.fori_loop(0, _NCHUNK // 2, loop_body, (), unroll=False)

    out_copy(_NCHUNK - 2, 0).wait()
    out_copy(_NCHUNK - 1, 1).wait()


@jax.jit
def kernel(x):
    ridx = jax.random.randint(
        jax.random.key(1), (x.shape[0],), 0, _NCOMB
    ).astype(jnp.int32)
    tab = jnp.asarray(
        np.concatenate([_ACOL, _BCOL, _TAILCOL[None, :]], axis=0)
    )
    run = pl.kernel(
        _body,
        out_type=jax.ShapeDtypeStruct((_NROWS, _NCOMB), jnp.float32),
        mesh=plsc.VectorSubcoreMesh(core_axis_name="c", subcore_axis_name="s"),
        scratch_types=[
            pltpu.VMEM((_CH, _NCOL), jnp.float32),
            pltpu.VMEM((_CH, _NCOL), jnp.float32),
            pltpu.VMEM((_CH, _NCOMB), jnp.float32),
            pltpu.VMEM((_CH, _NCOMB), jnp.float32),
            pltpu.VMEM((_ROWS_PER_W,), jnp.int32),
            pltpu.VMEM((2 * _NK + 1, _L), jnp.int32),
            pltpu.SemaphoreType.DMA,
            pltpu.SemaphoreType.DMA,
            pltpu.SemaphoreType.DMA,
            pltpu.SemaphoreType.DMA,
        ],
        compiler_params=pltpu.CompilerParams(needs_layout_passes=False),
    )
    return run(x, ridx, tab)


# speedup vs baseline: 2.4965x; 1.0025x over previous
import functools
import itertools

import jax
import jax.numpy as jnp
import numpy as np
from jax import lax
from jax.experimental import pallas as pl
from jax.experimental.pallas import tpu as pltpu
from jax.experimental.pallas import tpu_sc as plsc


def _combo_pairs():
    cola = [set(c) for c in itertools.combinations(range(8), 2)]
    pairs = []
    for i, si in enumerate(cola):
        for j, sj in enumerate(cola):
            if not si.intersection(sj):
                if [i, j] not in pairs and [j, i] not in pairs:
                    pairs.append([i, j])
    return np.array(sorted(pairs), dtype=np.int32)


_PAIRS = _combo_pairs()

_NROWS = 131072
_NCOL = 28
_NCOMB = 210
_NC = 2
_NS = 16
_NW = _NC * _NS
_ROWS_PER_W = _NROWS // _NW
_CH = 128
_NCHUNK = _ROWS_PER_W // _CH
_L = 16
_NK = 14

_PADDED = list(range(_NCOMB)) + list(range(_NK * _L - _NCOMB))
_ACOL = np.array(
    [[int(_PAIRS[_PADDED[k * _L + l], 0]) for l in range(_L)] for k in range(_NK)],
    dtype=np.int32,
)
_BCOL = np.array(
    [[int(_PAIRS[_PADDED[k * _L + l], 1]) for l in range(_L)] for k in range(_NK)],
    dtype=np.int32,
)
_TAILCOL = np.array(
    [(_NK - 1) * _L + l if l < 2 else 0 for l in range(_L)], dtype=np.int32
)


def _compute_chunk(xbuf, outbuf, ridxbuf, rbase, tabbuf):
    acols = tuple(tabbuf[k, :] for k in range(_NK))
    bcols = tuple(tabbuf[_NK + k, :] for k in range(_NK))
    tailcol = tabbuf[2 * _NK, :]
    tailmask = lax.iota(jnp.int32, _L) < 2
    zero = jnp.float32(0.0)
    bias = jnp.float32(1e-20)

    def row_body(r, carry):
        acols, bcols, tailcol = carry
        rvec = jnp.broadcast_to(r, (_L,)).astype(jnp.int32)
        vs = []
        smax = None
        for k in range(_NK):
            va = plsc.load_gather(xbuf, [rvec, acols[k]])
            vb = plsc.load_gather(xbuf, [rvec, bcols[k]])
            s = va + vb
            v = (jnp.abs(va - vb) + bias) / s
            smax = s if smax is None else jnp.maximum(smax, s)
            vs.append(v)
        for k in range(_NK - 1):
            outbuf[r, pl.ds(k * _L, _L)] = vs[k]
        plsc.store_scatter(outbuf, [rvec, tailcol], vs[_NK - 1], mask=tailmask)

        @pl.when(jnp.all(smax == zero))
        def _fix():
            colvec = plsc.load_gather(ridxbuf, [rvec + rbase])
            plsc.store_scatter(
                outbuf,
                [rvec, colvec],
                jnp.full((_L,), 1.0, jnp.float32),
                mask=lax.iota(jnp.int32, _L) < 1,
            )

        return carry

    lax.fori_loop(0, _CH, row_body, (acols, bcols, tailcol), unroll=4)


def _body(
    x_hbm, ridx_hbm, tab_hbm, out_hbm,
    xb0, xb1, ob0, ob1, ridxbuf, tabbuf,
    isem0, isem1, osem0, osem1,
):
    wid = lax.axis_index("s") * _NC + lax.axis_index("c")
    w_base = wid * _ROWS_PER_W
    pltpu.sync_copy(tab_hbm, tabbuf)
    pltpu.sync_copy(ridx_hbm.at[pl.ds(w_base, _ROWS_PER_W)], ridxbuf)

    xbufs = (xb0, xb1)
    obufs = (ob0, ob1)
    isems = (isem0, isem1)
    osems = (osem0, osem1)

    def in_copy(ci, p):
        return pltpu.make_async_copy(
            x_hbm.at[pl.ds(w_base + ci * _CH, _CH)], xbufs[p], isems[p]
        )

    def out_copy(ci, p):
        return pltpu.make_async_copy(
            obufs[p],
            out_hbm.at[pl.ds(w_base + ci * _CH, _CH)],
            osems[p],
        )

    in_copy(0, 0).start()
    in_copy(1, 1).start()

    def loop_body(k, _):
        for p in range(2):
            ci = 2 * k + p
            in_copy(ci, p).wait()

            @pl.when(ci >= 2)
            def _wait_out():
                out_copy(ci - 2, p).wait()

            _compute_chunk(xbufs[p], obufs[p], ridxbuf, ci * _CH, tabbuf)
            out_copy(ci, p).start()

            @pl.when(ci + 2 < _NCHUNK)
            def _prefetch():
                in_copy(ci + 2, p).start()
        return ()

    lax.fori_loop(0, _NCHUNK // 2, loop_body, (), unroll=False)

    out_copy(_NCHUNK - 2, 0).wait()
    out_copy(_NCHUNK - 1, 1).wait()


@jax.jit
def kernel(x):
    ridx = jax.random.randint(
        jax.random.key(1), (x.shape[0],), 0, _NCOMB
    ).astype(jnp.int32)
    tab = jnp.asarray(
        np.concatenate([_ACOL, _BCOL, _TAILCOL[None, :]], axis=0)
    )
    run = pl.kernel(
        _body,
        out_type=jax.ShapeDtypeStruct((_NROWS, _NCOMB), jnp.float32),
        mesh=plsc.VectorSubcoreMesh(core_axis_name="c", subcore_axis_name="s"),
        scratch_types=[
            pltpu.VMEM((_CH, _NCOL), jnp.float32),
            pltpu.VMEM((_CH, _NCOL), jnp.float32),
            pltpu.VMEM((_CH, _NCOMB), jnp.float32),
            pltpu.VMEM((_CH, _NCOMB), jnp.float32),
            pltpu.VMEM((_ROWS_PER_W,), jnp.int32),
            pltpu.VMEM((2 * _NK + 1, _L), jnp.int32),
            pltpu.SemaphoreType.DMA,
            pltpu.SemaphoreType.DMA,
            pltpu.SemaphoreType.DMA,
            pltpu.SemaphoreType.DMA,
        ],
        compiler_params=pltpu.CompilerParams(needs_layout_passes=False),
    )
    return run(x, ridx, tab)


# speedup vs baseline: 3.1951x; 1.2799x over previous
import functools
import itertools

import jax
import jax.numpy as jnp
import numpy as np
from jax import lax
from jax.experimental import pallas as pl
from jax.experimental.pallas import tpu as pltpu
from jax.experimental.pallas import tpu_sc as plsc


def _combo_pairs():
    cola = [set(c) for c in itertools.combinations(range(8), 2)]
    pairs = []
    for i, si in enumerate(cola):
        for j, sj in enumerate(cola):
            if not si.intersection(sj):
                if [i, j] not in pairs and [j, i] not in pairs:
                    pairs.append([i, j])
    return np.array(sorted(pairs), dtype=np.int32)


_PAIRS = _combo_pairs()

_NROWS = 131072
_NCOL = 28
_NCOMB = 210
_NC = 2
_NS = 16
_NW = _NC * _NS
_ROWS_PER_W = _NROWS // _NW
_CH = 128
_NCHUNK = _ROWS_PER_W // _CH
_L = 16
_NK = 14

_PADDED = list(range(_NCOMB)) + list(range(_NK * _L - _NCOMB))
_ACOL = np.array(
    [[int(_PAIRS[_PADDED[k * _L + l], 0]) for l in range(_L)] for k in range(_NK)],
    dtype=np.int32,
)
_BCOL = np.array(
    [[int(_PAIRS[_PADDED[k * _L + l], 1]) for l in range(_L)] for k in range(_NK)],
    dtype=np.int32,
)
_TAILCOL = np.array(
    [(_NK - 1) * _L + l if l < 2 else 0 for l in range(_L)], dtype=np.int32
)


def _compute_chunk(xbuf, outbuf, ridxbuf, rbase, tabbuf):
    acols = tuple(tabbuf[k, :] for k in range(_NK))
    bcols = tuple(tabbuf[_NK + k, :] for k in range(_NK))
    tailcol = tabbuf[2 * _NK, :]
    tailmask = lax.iota(jnp.int32, _L) < 2
    zero = jnp.float32(0.0)
    inf = jnp.float32(np.inf)

    def row_body(r, carry):
        acols, bcols, tailcol, zmin = carry
        rvec = jnp.broadcast_to(r, (_L,)).astype(jnp.int32)
        vs = []
        for k in range(_NK):
            va = plsc.load_gather(xbuf, [rvec, acols[k]])
            vb = plsc.load_gather(xbuf, [rvec, bcols[k]])
            v = jnp.abs(va - vb) / (va + vb)
            vs.append(v)
        for k in range(_NK - 1):
            outbuf[r, pl.ds(k * _L, _L)] = vs[k]
        plsc.store_scatter(outbuf, [rvec, tailcol], vs[_NK - 1], mask=tailmask)
        v1 = xbuf[r, pl.ds(0, _L)]
        v2 = xbuf[r, pl.ds(_NCOL - _L, _L)]
        zmin = jnp.minimum(zmin, jnp.minimum(v1, v2))
        return (acols, bcols, tailcol, zmin)

    carry = lax.fori_loop(
        0,
        _CH,
        row_body,
        (acols, bcols, tailcol, jnp.full((_L,), 1.0, jnp.float32)),
        unroll=2,
    )
    zmin = carry[3]

    @pl.when(jnp.any(zmin == zero))
    def _rescan():
        def fix_row(r, _):
            rvec = jnp.broadcast_to(r, (_L,)).astype(jnp.int32)
            vs = []
            allnan = None
            for k in range(_NK):
                va = plsc.load_gather(xbuf, [rvec, acols[k]])
                vb = plsc.load_gather(xbuf, [rvec, bcols[k]])
                s = va + vb
                z = s == zero
                v = jnp.where(z, inf, jnp.abs(va - vb) / s)
                allnan = z if allnan is None else (allnan & z)
                vs.append(v)
            for k in range(_NK - 1):
                outbuf[r, pl.ds(k * _L, _L)] = vs[k]
            plsc.store_scatter(
                outbuf, [rvec, tailcol], vs[_NK - 1], mask=tailmask
            )

            @pl.when(jnp.all(allnan))
            def _fix():
                colvec = plsc.load_gather(ridxbuf, [rvec + rbase])
                plsc.store_scatter(
                    outbuf,
                    [rvec, colvec],
                    jnp.full((_L,), 1.0, jnp.float32),
                    mask=lax.iota(jnp.int32, _L) < 1,
                )

            return ()

        lax.fori_loop(0, _CH, fix_row, (), unroll=False)


def _body(
    x_hbm, ridx_hbm, tab_hbm, out_hbm,
    xb0, xb1, ob0, ob1, ridxbuf, tabbuf,
    isem0, isem1, osem0, osem1,
):
    wid = lax.axis_index("s") * _NC + lax.axis_index("c")
    w_base = wid * _ROWS_PER_W
    pltpu.sync_copy(tab_hbm, tabbuf)
    pltpu.sync_copy(ridx_hbm.at[pl.ds(w_base, _ROWS_PER_W)], ridxbuf)

    xbufs = (xb0, xb1)
    obufs = (ob0, ob1)
    isems = (isem0, isem1)
    osems = (osem0, osem1)

    def in_copy(ci, p):
        return pltpu.make_async_copy(
            x_hbm.at[pl.ds(w_base + ci * _CH, _CH)], xbufs[p], isems[p]
        )

    def out_copy(ci, p):
        return pltpu.make_async_copy(
            obufs[p],
            out_hbm.at[pl.ds(w_base + ci * _CH, _CH)],
            osems[p],
        )

    in_copy(0, 0).start()
    in_copy(1, 1).start()

    def loop_body(k, _):
        for p in range(2):
            ci = 2 * k + p
            in_copy(ci, p).wait()

            @pl.when(ci >= 2)
            def _wait_out():
                out_copy(ci - 2, p).wait()

            _compute_chunk(xbufs[p], obufs[p], ridxbuf, ci * _CH, tabbuf)
            out_copy(ci, p).start()

            @pl.when(ci + 2 < _NCHUNK)
            def _prefetch():
                in_copy(ci + 2, p).start()
        return ()

    lax.fori_loop(0, _NCHUNK // 2, loop_body, (), unroll=False)

    out_copy(_NCHUNK - 2, 0).wait()
    out_copy(_NCHUNK - 1, 1).wait()


@jax.jit
def kernel(x):
    ridx = jax.random.randint(
        jax.random.key(1), (x.shape[0],), 0, _NCOMB
    ).astype(jnp.int32)
    tab = jnp.asarray(
        np.concatenate([_ACOL, _BCOL, _TAILCOL[None, :]], axis=0)
    )
    run = pl.kernel(
        _body,
        out_type=jax.ShapeDtypeStruct((_NROWS, _NCOMB), jnp.float32),
        mesh=plsc.VectorSubcoreMesh(core_axis_name="c", subcore_axis_name="s"),
        scratch_types=[
            pltpu.VMEM((_CH, _NCOL), jnp.float32),
            pltpu.VMEM((_CH, _NCOL), jnp.float32),
            pltpu.VMEM((_CH, _NCOMB), jnp.float32),
            pltpu.VMEM((_CH, _NCOMB), jnp.float32),
            pltpu.VMEM((_ROWS_PER_W,), jnp.int32),
            pltpu.VMEM((2 * _NK + 1, _L), jnp.int32),
            pltpu.SemaphoreType.DMA,
            pltpu.SemaphoreType.DMA,
            pltpu.SemaphoreType.DMA,
            pltpu.SemaphoreType.DMA,
        ],
        compiler_params=pltpu.CompilerParams(needs_layout_passes=False),
    )
    return run(x, ridx, tab)
